# Initial kernel scaffold; baseline (speedup 1.0000x reference)
import jax
import jax.numpy as jnp
from jax.experimental import pallas as pl


def kernel(x, codebooks):
    raise NotImplementedError("write your pallas kernel here")


# speedup vs baseline: 1.5708x; 1.5708x over previous
import functools

import jax
import jax.numpy as jnp
from jax import lax
from jax.experimental import pallas as pl
from jax.experimental.pallas import tpu as pltpu
from jax.experimental.pallas import tpu_sc as plsc

NUM_Q = 4
K = 8192
D = 64
T = 8192

TB = 512

NC = 2
NS = 16
NW = NC * NS
TPW = T // NW
IDX_ROWS = T // 128
ROWS_PER_W = IDX_ROWS // NW


def _argmin_body(r_ref, cbt_ref, rn_ref, cn_ref, out_ref):
    r2 = (r_ref[...] * 2.0).astype(jnp.bfloat16)
    m2 = jax.lax.dot_general(
        r2, cbt_ref[...],
        dimension_numbers=(((1,), (0,)), ((), ())),
        preferred_element_type=jnp.float32,
    )
    s2 = (rn_ref[...] - m2) + cn_ref[...]
    ind = jnp.argmin(s2, axis=1).astype(jnp.int32)
    out_ref[...] = ind.reshape(TB, 1)


def _tc_argmin(r, cbt_bf16, rn, cn):
    grid = (T // TB,)
    return pl.pallas_call(
        _argmin_body,
        grid=grid,
        in_specs=[
            pl.BlockSpec((TB, D), lambda i: (i, 0)),
            pl.BlockSpec((D, K), lambda i: (0, 0)),
            pl.BlockSpec((TB, 1), lambda i: (i, 0)),
            pl.BlockSpec((1, K), lambda i: (0, 0)),
        ],
        out_specs=pl.BlockSpec((TB, 1), lambda i: (i, 0)),
        out_shape=jax.ShapeDtypeStruct((T, 1), jnp.int32),
        compiler_params=pltpu.CompilerParams(
            dimension_semantics=("arbitrary",),
        ),
    )(r, cbt_bf16, rn, cn)


def _sc_update_body(cb_hbm, idx_hbm, r_hbm, out_hbm, idx_v, rows_v, r_v, o_v, sem):
    wid = lax.axis_index("s") * NC + lax.axis_index("c")
    pltpu.sync_copy(idx_hbm.at[pl.ds(wid * ROWS_PER_W, ROWS_PER_W)], idx_v)
    for j in range(ROWS_PER_W):
        pltpu.async_copy(
            cb_hbm.at[idx_v.at[j]], rows_v.at[pl.ds(j * 128, 128)], sem
        ).wait()
    pltpu.sync_copy(r_hbm.at[pl.ds(wid * TPW, TPW)], r_v)

    def body(i, _):
        q_row = rows_v.at[i]
        r_row = r_v.at[i]
        o_row = o_v.at[i]
        for j in range(D // 16):
            sl = pl.ds(j * 16, 16)
            qv = q_row[sl]
            rv = r_row[sl]
            q2 = rv + (qv - rv)
            o_row[sl] = rv - q2
        return 0

    lax.fori_loop(0, TPW, body, 0)
    pltpu.sync_copy(o_v, out_hbm.at[pl.ds(wid * TPW, TPW)])


@functools.cache
def _sc_update():
    return pl.kernel(
        _sc_update_body,
        out_type=jax.ShapeDtypeStruct((T, D), jnp.float32),
        mesh=plsc.VectorSubcoreMesh(core_axis_name="c", subcore_axis_name="s"),
        scratch_types=[
            pltpu.VMEM((ROWS_PER_W, 128), jnp.int32),
            pltpu.VMEM((TPW, 128), jnp.float32),
            pltpu.VMEM((TPW, D), jnp.float32),
            pltpu.VMEM((TPW, D), jnp.float32),
            pltpu.SemaphoreType.DMA,
        ],
    )


def kernel(x, codebooks):
    r = x.reshape(T, D)
    cbt = codebooks.astype(jnp.bfloat16).transpose(0, 2, 1)
    cn = jnp.sum(codebooks**2, axis=-1)
    cbp = jnp.pad(codebooks, ((0, 0), (0, 0), (0, 128 - D)))
    inds = []
    for q in range(NUM_Q):
        rn = jnp.sum(r**2, axis=-1, keepdims=True)
        ind = _tc_argmin(r, cbt[q], rn, cn[q].reshape(1, K))
        r = _sc_update()(cbp[q], ind.reshape(IDX_ROWS, 128), r)
        inds.append(ind.reshape(8, 1024))
    quantized_out = x - r.reshape(x.shape)
    indices = jnp.stack(inds, axis=-1)
    return quantized_out, indices
